# baseline (speedup 1.0000x reference)
import functools

import jax
import jax.numpy as jnp
from jax import lax
from jax.experimental import pallas as pl
from jax.experimental.pallas import tpu as pltpu
from jax.experimental.pallas import tpu_sc as plsc

N = 10000
E = 320000
H = 64

NC = 2
NS = 16
NW = NC * NS

GRP = 128
SUP = 1024
SUPS_PER_PAIR = 20
SUP0 = 14
SUP1 = SUPS_PER_PAIR - SUP0
E_PAD = NS * SUPS_PER_PAIR * SUP
CHUNK = 512
NSUB = SUP // CHUNK
NGRP = CHUNK // GRP

N_PAD = 10240
RPT = N_PAD // NS

_f32 = jnp.float32


def _lrelu(x):
    return jnp.maximum(x, 0.01 * x)


def _prep_node_body(nf, Wn, bn, Wi, bi, We, be, node2_o, nodeP_o, Wc_o, bc_o):
    node2 = jnp.dot(nf[...], Wn[...], preferred_element_type=_f32) + bn[...]
    node2_o[...] = node2
    Wi_v = Wi[...]
    nodeP_o[...] = jnp.dot(node2, Wi_v[:H], preferred_element_type=_f32)
    Wc_o[...] = jnp.dot(We[...], Wi_v[H:], preferred_element_type=_f32)
    bc_o[...] = jnp.dot(be[...], Wi_v[H:], preferred_element_type=_f32) + bi[...]


EQK = 8
EQR = CHUNK // 2


def _edgeq_body(*refs):
    ins, Wc, bc, out = refs[:2 * EQK], refs[2 * EQK], refs[2 * EQK + 1], refs[-1]
    cat = jnp.concatenate([r[...] for r in ins], axis=0)
    q = jnp.dot(cat, Wc[...], preferred_element_type=_f32) + bc[...]
    for j in range(EQK):
        out[j * EQR:(j + 1) * EQR, 0:H] = q[2 * j * EQR:(2 * j + 1) * EQR]
        out[j * EQR:(j + 1) * EQR, H:2 * H] = q[(2 * j + 1) * EQR:(2 * j + 2) * EQR]


def _mix_body(pp, W, b, out):
    s = pp[0] + pp[1]
    out[...] = jnp.dot(s, W[...], preferred_element_type=_f32) + b[...]


def _final_body(node2, pp, Wa, ba, Wl1, bl1, Wl2, bl2, out):
    agg = pp[0, :N] + pp[1, :N]
    Wa_v = Wa[...]
    h = (jnp.dot(node2[...], Wa_v[:H], preferred_element_type=_f32)
         + jnp.dot(agg, Wa_v[H:], preferred_element_type=_f32) + ba[...])
    h = _lrelu(h)
    g = jnp.sum(h, axis=0, keepdims=True)
    y = _lrelu(jnp.dot(g, Wl1[...], preferred_element_type=_f32) + bl1[...])
    out[...] = jnp.dot(y, Wl2[...], preferred_element_type=_f32) + bl2[...]


def _sc_pass_body(paired, t_hbm, src2d, dst2d, base_hbm,
                  partial_o, h_o, acc_s, src_v, dst_v, base_v, gath_v,
                  isem, bsem, gsem, ssem, hsem):
    cid = lax.axis_index("c")
    sid = lax.axis_index("s")
    sup_base = sid * SUPS_PER_PAIR + cid * SUP0
    nsup = lax.select(cid == 0, SUP0, SUP1)

    def zrow(r, cc):
        for c4 in range(H // 16):
            gath_v[r, pl.ds(c4 * 16, 16)] = jnp.zeros((16,), _f32)
        return cc

    lax.fori_loop(0, CHUNK, zrow, 0)
    pltpu.sync_copy(gath_v, acc_s.at[pl.ds(sid * RPT, CHUNK), :])
    pltpu.sync_copy(gath_v.at[pl.ds(0, RPT - CHUNK), :],
                    acc_s.at[pl.ds(sid * RPT + CHUNK, RPT - CHUNK), :])
    plsc.subcore_barrier()

    def superchunk(i, carry):
        e00 = (sup_base + i) * SUP
        r0 = (sup_base + i) * (SUP // GRP)
        di = pltpu.async_copy(src2d.at[pl.ds(r0, SUP // GRP), :], src_v, isem)
        di2 = pltpu.async_copy(dst2d.at[pl.ds(r0, SUP // GRP), :], dst_v, isem)
        idx_waited = [False]
        for s in range(NSUB):
            e0 = e00 + s * CHUNK
            if paired:
                db = pltpu.async_copy(
                    base_hbm.at[pl.ds(e0 // 2, CHUNK // 2), :], base_v, bsem)
            else:
                db = pltpu.async_copy(
                    base_hbm.at[pl.ds(e0, CHUNK), :], base_v, bsem)
            if not idx_waited[0]:
                di.wait()
                di2.wait()
                idx_waited[0] = True
            dg = [
                pltpu.async_copy(t_hbm.at[src_v.at[s * NGRP + j]],
                                 gath_v.at[pl.ds(j * GRP, GRP), :], gsem)
                for j in range(NGRP)
            ]
            db.wait()
            for d in dg:
                d.wait()

            if paired:
                def row(r2, cc):
                    for c8 in range(2 * H // 16):
                        gro = (c8 // 4) * (CHUNK // 2)
                        gc = (c8 % 4) * 16
                        x = (base_v[r2, pl.ds(c8 * 16, 16)]
                             + gath_v[gro + r2, pl.ds(gc, 16)])
                        gath_v[gro + r2, pl.ds(gc, 16)] = jnp.maximum(x, 0.01 * x)
                    return cc

                lax.fori_loop(0, CHUNK // 2, row, 0)
            else:
                def row(r, cc):
                    for c4 in range(H // 16):
                        x = (base_v[r, pl.ds(c4 * 16, 16)]
                             + gath_v[r, pl.ds(c4 * 16, 16)])
                        gath_v[r, pl.ds(c4 * 16, 16)] = jnp.maximum(x, 0.01 * x)
                    return cc

                lax.fori_loop(0, CHUNK, row, 0)
            if paired:
                dh = pltpu.async_copy(gath_v, h_o.at[pl.ds(e0, CHUNK), :],
                                      hsem)
            ds = [
                pltpu.async_copy(gath_v.at[pl.ds(j * GRP, GRP), :],
                                 acc_s.at[dst_v.at[s * NGRP + j]], ssem,
                                 add=True)
                for j in range(NGRP)
            ]
            if paired:
                dh.wait()
            for d in ds:
                d.wait()
        return carry

    lax.fori_loop(0, nsup, superchunk, 0)
    plsc.subcore_barrier()
    pltpu.sync_copy(acc_s.at[pl.ds(sid * RPT, RPT), :],
                    partial_o.at[cid, pl.ds(sid * RPT, RPT), :])


def _make_sc_pass(paired):
    mesh = plsc.VectorSubcoreMesh(core_axis_name="c", subcore_axis_name="s")
    out_type = [jax.ShapeDtypeStruct((NC, N_PAD, H), _f32)]
    if paired:
        out_type.append(jax.ShapeDtypeStruct((E_PAD, H), _f32))
    else:
        out_type.append(jax.ShapeDtypeStruct((8, H), _f32))
    base_shape = (CHUNK // 2, 2 * H) if paired else (CHUNK, H)
    return pl.kernel(
        functools.partial(_sc_pass_body, paired),
        out_type=tuple(out_type),
        mesh=mesh,
        scratch_types=(
            [pltpu.VMEM_SHARED((N_PAD, H), _f32)]
            + [pltpu.VMEM((SUP // GRP, GRP), jnp.int32)] * 2
            + [pltpu.VMEM(base_shape, _f32)]
            + [pltpu.VMEM((CHUNK, H), _f32)]
            + [pltpu.SemaphoreType.DMA] * 5
        ),
        compiler_params=pltpu.CompilerParams(use_tc_tiling_on_sc=False),
        name="sc_pass_h" if paired else "sc_pass",
    )


_sc_pass0 = _make_sc_pass(True)
_sc_pass = _make_sc_pass(False)


def kernel(node_feats, edge_feats, edge_index, W_node, b_node, W_edge, b_edge,
           W_init, b_init, W_h, b_h, W_a, b_a, W_l1, b_l1, W_l2, b_l2):
    src = edge_index[0]
    dst = edge_index[1]
    pad = E_PAD - E
    src2d = jnp.concatenate([src, jnp.zeros((pad,), jnp.int32)]).reshape(-1, GRP)
    dst2d = jnp.concatenate([dst, jnp.full((pad,), N, jnp.int32)]).reshape(-1, GRP)

    bn = b_node.reshape(1, H)
    bi = b_init.reshape(1, H)
    be = b_edge.reshape(1, H)
    ba = b_a.reshape(1, H)
    bl1 = b_l1.reshape(1, H)
    bl2 = b_l2.reshape(1, -1)
    ED = edge_feats.shape[1]

    node2, nodeP, Wc, bc = pl.pallas_call(
        _prep_node_body,
        out_shape=(
            jax.ShapeDtypeStruct((N, H), _f32),
            jax.ShapeDtypeStruct((N, H), _f32),
            jax.ShapeDtypeStruct((ED, H), _f32),
            jax.ShapeDtypeStruct((1, H), _f32),
        ),
    )(node_feats, W_node, bn, W_init, bi, W_edge, be)

    nreal = E // EQR
    in_specs = []
    for j in range(2 * EQK):
        in_specs.append(pl.BlockSpec(
            (EQR, ED),
            functools.partial(
                lambda i, jj: (jnp.minimum(2 * EQK * i + jj, nreal - 1), 0),
                jj=j)))
    in_specs.append(pl.BlockSpec((ED, H), lambda i: (0, 0)))
    in_specs.append(pl.BlockSpec((1, H), lambda i: (0, 0)))
    edgeQ = pl.pallas_call(
        _edgeq_body,
        grid=(E_PAD // (EQK * CHUNK),),
        in_specs=in_specs,
        out_specs=pl.BlockSpec((EQK * EQR, 2 * H), lambda i: (i, 0)),
        out_shape=jax.ShapeDtypeStruct((E_PAD // 2, 2 * H), _f32),
    )(*([edge_feats] * (2 * EQK)), Wc, bc)

    partial, h0 = _sc_pass0(nodeP, src2d, dst2d, edgeQ)

    mix = pl.pallas_call(
        _mix_body,
        out_shape=jax.ShapeDtypeStruct((N_PAD, H), _f32),
    )
    for l in range(W_h.shape[0]):
        t = mix(partial, W_h[l], b_h[l].reshape(1, H))
        partial, _ = _sc_pass(t, src2d, dst2d, h0)

    out = pl.pallas_call(
        _final_body,
        out_shape=jax.ShapeDtypeStruct((1, W_l2.shape[1]), _f32),
    )(node2, partial, W_a, ba, W_l1, bl1, W_l2, bl2)
    return out


# speedup vs baseline: 1.0854x; 1.0854x over previous
import functools

import jax
import jax.numpy as jnp
from jax import lax
from jax.experimental import pallas as pl
from jax.experimental.pallas import tpu as pltpu
from jax.experimental.pallas import tpu_sc as plsc

N = 10000
E = 320000
H = 64

NC = 2
NS = 16
NW = NC * NS

GRP = 128
SUP = 1024
SUPS_PER_PAIR = 20
SUP0 = 14
SUP1 = SUPS_PER_PAIR - SUP0
E_PAD = NS * SUPS_PER_PAIR * SUP
CHUNK = 512
NSUB = SUP // CHUNK
NGRP = CHUNK // GRP

N_PAD = 10240
RPT = N_PAD // NS

_f32 = jnp.float32


def _lrelu(x):
    return jnp.maximum(x, 0.01 * x)


_HI = lax.Precision.HIGHEST


def _prep_node_body(nf, Wn, bn, Wi, bi, We, be, node2_o, nodeP_o, W8_o, b8_o):
    node2 = jnp.dot(nf[...], Wn[...], preferred_element_type=_f32,
                    precision=_HI) + bn[...]
    node2_o[...] = node2
    Wi_v = Wi[...]
    nodeP_o[...] = jnp.dot(node2, Wi_v[:H], preferred_element_type=_f32,
                           precision=_HI)
    Wc = jnp.dot(We[...], Wi_v[H:], preferred_element_type=_f32, precision=_HI)
    ED = Wc.shape[0]
    bc = jnp.dot(be[...], Wi_v[H:], preferred_element_type=_f32,
                 precision=_HI) + bi[...]
    W8_o[...] = jnp.zeros((8 * ED, 8 * H), _f32)
    for j in range(8):
        W8_o[j * ED:(j + 1) * ED, j * H:(j + 1) * H] = Wc
        b8_o[:, j * H:(j + 1) * H] = bc


def _edgeq_body(efp, W8, b8, out):
    out[...] = jnp.dot(efp[...], W8[...], preferred_element_type=_f32) + b8[...]


def _mix_body(pp, W, b, out):
    s = pp[0] + pp[1]
    out[...] = jnp.dot(s, W[...], preferred_element_type=_f32,
                       precision=_HI) + b[...]


def _final_body(node2, pp, Wa, ba, Wl1, bl1, Wl2, bl2, out):
    agg = pp[0, :N] + pp[1, :N]
    Wa_v = Wa[...]
    h = (jnp.dot(node2[...], Wa_v[:H], preferred_element_type=_f32)
         + jnp.dot(agg, Wa_v[H:], preferred_element_type=_f32) + ba[...])
    h = _lrelu(h)
    g = jnp.sum(h, axis=0, keepdims=True)
    y = _lrelu(jnp.dot(g, Wl1[...], preferred_element_type=_f32) + bl1[...])
    out[...] = jnp.dot(y, Wl2[...], preferred_element_type=_f32) + bl2[...]


def _sc_pass_body(paired, t_hbm, src2d, dst2d, base_hbm,
                  partial_o, h_o, acc_s, src_v, dst_v, base_v, gath_v,
                  isem, bsem, gsem, ssem, hsem):
    cid = lax.axis_index("c")
    sid = lax.axis_index("s")
    sup_base = sid * SUPS_PER_PAIR + cid * SUP0
    nsup = lax.select(cid == 0, SUP0, SUP1)

    def zrow(r, cc):
        for c4 in range(H // 16):
            gath_v[r, pl.ds(c4 * 16, 16)] = jnp.zeros((16,), _f32)
        return cc

    lax.fori_loop(0, CHUNK, zrow, 0)
    pltpu.sync_copy(gath_v, acc_s.at[pl.ds(sid * RPT, CHUNK), :])
    pltpu.sync_copy(gath_v.at[pl.ds(0, RPT - CHUNK), :],
                    acc_s.at[pl.ds(sid * RPT + CHUNK, RPT - CHUNK), :])
    plsc.subcore_barrier()

    def superchunk(i, carry):
        e00 = (sup_base + i) * SUP
        r0 = (sup_base + i) * (SUP // GRP)
        di = pltpu.async_copy(src2d.at[pl.ds(r0, SUP // GRP), :], src_v, isem)
        di2 = pltpu.async_copy(dst2d.at[pl.ds(r0, SUP // GRP), :], dst_v, isem)
        idx_waited = [False]
        for s in range(NSUB):
            e0 = e00 + s * CHUNK
            if paired:
                db = pltpu.async_copy(
                    base_hbm.at[pl.ds(e0 // 8, CHUNK // 8), :], base_v, bsem)
            else:
                db = pltpu.async_copy(
                    base_hbm.at[pl.ds(e0, CHUNK), :], base_v, bsem)
            if not idx_waited[0]:
                di.wait()
                di2.wait()
                idx_waited[0] = True
            dg = [
                pltpu.async_copy(t_hbm.at[src_v.at[s * NGRP + j]],
                                 gath_v.at[pl.ds(j * GRP, GRP), :], gsem)
                for j in range(NGRP)
            ]
            db.wait()
            for d in dg:
                d.wait()

            if paired:
                def row(r8, cc):
                    for c32 in range(8 * H // 16):
                        gr = 8 * r8 + (c32 // 4)
                        gc = (c32 % 4) * 16
                        x = (base_v[r8, pl.ds(c32 * 16, 16)]
                             + gath_v[gr, pl.ds(gc, 16)])
                        gath_v[gr, pl.ds(gc, 16)] = jnp.maximum(x, 0.01 * x)
                    return cc

                lax.fori_loop(0, CHUNK // 8, row, 0)
            else:
                def row(r, cc):
                    for c4 in range(H // 16):
                        x = (base_v[r, pl.ds(c4 * 16, 16)]
                             + gath_v[r, pl.ds(c4 * 16, 16)])
                        gath_v[r, pl.ds(c4 * 16, 16)] = jnp.maximum(x, 0.01 * x)
                    return cc

                lax.fori_loop(0, CHUNK, row, 0)
            if paired:
                dh = pltpu.async_copy(gath_v, h_o.at[pl.ds(e0, CHUNK), :],
                                      hsem)
            ds = [
                pltpu.async_copy(gath_v.at[pl.ds(j * GRP, GRP), :],
                                 acc_s.at[dst_v.at[s * NGRP + j]], ssem,
                                 add=True)
                for j in range(NGRP)
            ]
            if paired:
                dh.wait()
            for d in ds:
                d.wait()
        return carry

    lax.fori_loop(0, nsup, superchunk, 0)
    plsc.subcore_barrier()
    pltpu.sync_copy(acc_s.at[pl.ds(sid * RPT, RPT), :],
                    partial_o.at[cid, pl.ds(sid * RPT, RPT), :])


def _make_sc_pass(paired):
    mesh = plsc.VectorSubcoreMesh(core_axis_name="c", subcore_axis_name="s")
    out_type = [jax.ShapeDtypeStruct((NC, N_PAD, H), _f32)]
    if paired:
        out_type.append(jax.ShapeDtypeStruct((E_PAD, H), _f32))
    else:
        out_type.append(jax.ShapeDtypeStruct((8, H), _f32))
    base_shape = (CHUNK // 8, 8 * H) if paired else (CHUNK, H)
    return pl.kernel(
        functools.partial(_sc_pass_body, paired),
        out_type=tuple(out_type),
        mesh=mesh,
        scratch_types=(
            [pltpu.VMEM_SHARED((N_PAD, H), _f32)]
            + [pltpu.VMEM((SUP // GRP, GRP), jnp.int32)] * 2
            + [pltpu.VMEM(base_shape, _f32)]
            + [pltpu.VMEM((CHUNK, H), _f32)]
            + [pltpu.SemaphoreType.DMA] * 5
        ),
        compiler_params=pltpu.CompilerParams(use_tc_tiling_on_sc=False),
        name="sc_pass_h" if paired else "sc_pass",
    )


_sc_pass0 = _make_sc_pass(True)
_sc_pass = _make_sc_pass(False)


def kernel(node_feats, edge_feats, edge_index, W_node, b_node, W_edge, b_edge,
           W_init, b_init, W_h, b_h, W_a, b_a, W_l1, b_l1, W_l2, b_l2):
    src = edge_index[0]
    dst = edge_index[1]
    pad = E_PAD - E
    src2d = jnp.concatenate([src, jnp.zeros((pad,), jnp.int32)]).reshape(-1, GRP)
    dst2d = jnp.concatenate([dst, jnp.full((pad,), N, jnp.int32)]).reshape(-1, GRP)

    bn = b_node.reshape(1, H)
    bi = b_init.reshape(1, H)
    be = b_edge.reshape(1, H)
    ba = b_a.reshape(1, H)
    bl1 = b_l1.reshape(1, H)
    bl2 = b_l2.reshape(1, -1)
    ED = edge_feats.shape[1]
    efp = edge_feats.reshape(E // 8, 8 * ED)

    node2, nodeP, W8, b8 = pl.pallas_call(
        _prep_node_body,
        out_shape=(
            jax.ShapeDtypeStruct((N, H), _f32),
            jax.ShapeDtypeStruct((N, H), _f32),
            jax.ShapeDtypeStruct((8 * ED, 8 * H), _f32),
            jax.ShapeDtypeStruct((1, 8 * H), _f32),
        ),
    )(node_feats, W_node, bn, W_init, bi, W_edge, be)

    EBX = 320
    nreal = (E // 8) // EBX
    edgeQ = pl.pallas_call(
        _edgeq_body,
        grid=((E_PAD // 8) // EBX,),
        in_specs=[
            pl.BlockSpec((EBX, 8 * ED), lambda i: (jnp.minimum(i, nreal - 1), 0)),
            pl.BlockSpec((8 * ED, 8 * H), lambda i: (0, 0)),
            pl.BlockSpec((1, 8 * H), lambda i: (0, 0)),
        ],
        out_specs=pl.BlockSpec((EBX, 8 * H), lambda i: (i, 0)),
        out_shape=jax.ShapeDtypeStruct((E_PAD // 8, 8 * H), _f32),
    )(efp, W8, b8)

    partial, h0 = _sc_pass0(nodeP, src2d, dst2d, edgeQ)

    mix = pl.pallas_call(
        _mix_body,
        out_shape=jax.ShapeDtypeStruct((N_PAD, H), _f32),
    )
    for l in range(W_h.shape[0]):
        t = mix(partial, W_h[l], b_h[l].reshape(1, H))
        partial, _ = _sc_pass(t, src2d, dst2d, h0)

    out = pl.pallas_call(
        _final_body,
        out_shape=jax.ShapeDtypeStruct((1, W_l2.shape[1]), _f32),
    )(node2, partial, W_a, ba, W_l1, bl1, W_l2, bl2)
    return out
